# baseline (speedup 1.0000x reference)
import functools

import jax
import jax.numpy as jnp
from jax import lax
from jax.experimental import pallas as pl
from jax.experimental.pallas import tpu as pltpu
from jax.experimental.pallas import tpu_sc as plsc

_B, _T, _K, _CD, _DIM = 8, 4096, 8192, 256, 1024
_CDP = _CD // 2
_N = _B * _T
_NW = 32
_BH = _B // 2
_NH = _BH * _T
_BPW = _NH // _NW
_CHUNK = 128
_NCHUNK = _BPW // _CHUNK
_PB = 4096


def _pack_body(cb_ref, o_ref):
    u = lax.bitcast_convert_type(cb_ref[...], jnp.int32)
    r = (u + jnp.int32(32767) + ((u >> 16) & jnp.int32(1))) >> 16
    o_ref[...] = (r[:, :_CDP] & jnp.int32(0xFFFF)) | (r[:, _CDP:] << 16)


def _sc_gather_body(table_hbm, idx_hbm, out_hbm, idx_v, rows0, rows1, sem0, sem1):
    wid = lax.axis_index("s") * 2 + lax.axis_index("c")
    base = wid * _BPW
    row = base // _T
    col = base % _T
    pltpu.sync_copy(idx_hbm.at[row, pl.ds(col, _BPW)], idx_v)
    bufs = (rows0, rows1)
    sems = (sem0, sem1)
    cp = pltpu.async_copy(table_hbm.at[idx_v.at[pl.ds(0, _CHUNK)]], bufs[0], sems[0])
    for c in range(_NCHUNK):
        cp.wait()
        if c + 1 < _NCHUNK:
            cp = pltpu.async_copy(
                table_hbm.at[idx_v.at[pl.ds((c + 1) * _CHUNK, _CHUNK)]],
                bufs[(c + 1) % 2],
                sems[(c + 1) % 2],
            )
        pltpu.sync_copy(bufs[c % 2], out_hbm.at[pl.ds(base + c * _CHUNK, _CHUNK)])


_sc_gather = functools.partial(
    pl.kernel,
    mesh=plsc.VectorSubcoreMesh(core_axis_name="c", subcore_axis_name="s"),
    out_type=jax.ShapeDtypeStruct((_NH, _CDP), jnp.int32),
    scratch_types=[
        pltpu.VMEM((_BPW,), jnp.int32),
        pltpu.VMEM((_CHUNK, _CDP), jnp.int32),
        pltpu.VMEM((_CHUNK, _CDP), jnp.int32),
        pltpu.SemaphoreType.DMA,
        pltpu.SemaphoreType.DMA,
    ],
)(_sc_gather_body)


_TT = 4096


def _mm_body(x_ref, w_ref, b_ref, *rest):
    o_ref = rest[-1]
    x = x_ref[0]
    xl = lax.bitcast_convert_type(x << 16, jnp.float32)
    xh = lax.bitcast_convert_type(x & jnp.int32(-65536), jnp.float32)
    x2 = jnp.concatenate([xl, xh], axis=1)
    acc = lax.dot_general(
        w_ref[...], x2, (((1,), (1,)), ((), ())), preferred_element_type=jnp.float32
    )
    o_ref[0] = acc + b_ref[...]


def _mm_half(gathered_h, W, b2, half, prev=None):
    in_specs = [
        pl.BlockSpec((1, _T, _CDP), lambda bb: (bb, 0, 0)),
        pl.BlockSpec((_DIM, _CD), lambda bb: (0, 0)),
        pl.BlockSpec((_DIM, 1), lambda bb: (0, 0)),
    ]
    args = [gathered_h.reshape(_BH, _T, _CDP), W, b2]
    kwargs = {}
    if prev is not None:
        in_specs.append(pl.BlockSpec(memory_space=pl.ANY))
        args.append(prev)
        kwargs["input_output_aliases"] = {3: 0}
    base = half * _BH
    return pl.pallas_call(
        _mm_body,
        grid=(_BH,),
        in_specs=in_specs,
        out_specs=pl.BlockSpec((1, _DIM, _T), lambda bb: (bb + base, 0, 0)),
        out_shape=jax.ShapeDtypeStruct((_B, _DIM, _T), jnp.float32),
        **kwargs,
    )(*args)


def kernel(indices, codebook, W, b):
    packed = pl.pallas_call(
        _pack_body,
        grid=(_K // _PB,),
        in_specs=[pl.BlockSpec((_PB, _CD), lambda i: (i, 0))],
        out_specs=pl.BlockSpec((_PB, _CDP), lambda i: (i, 0)),
        out_shape=jax.ShapeDtypeStruct((_K, _CDP), jnp.int32),
    )(codebook)
    b2 = b.reshape(_DIM, 1)
    g0 = _sc_gather(packed, indices[:_BH])
    g1 = _sc_gather(packed, indices[_BH:])
    out = _mm_half(g0, W, b2, 0)
    out = _mm_half(g1, W, b2, 1, prev=out)
    return out


# speedup vs baseline: 1.0788x; 1.0788x over previous
import functools

import jax
import jax.numpy as jnp
from jax import lax
from jax.experimental import pallas as pl
from jax.experimental.pallas import tpu as pltpu
from jax.experimental.pallas import tpu_sc as plsc

_B, _T, _K, _CD, _DIM = 8, 4096, 8192, 256, 1024
_CDP = _CD // 2
_N = _B * _T
_NW = 32
_BPW = _N // _NW
_CHUNK = 128
_NCHUNK = _BPW // _CHUNK
_NBUF = 3
_PB = 4096
_TT = 4096


def _pack_body(cb_ref, o_ref):
    u = lax.bitcast_convert_type(cb_ref[...], jnp.int32)
    r = (u + jnp.int32(32767) + ((u >> 16) & jnp.int32(1))) >> 16
    o_ref[...] = (r[:, :_CDP] & jnp.int32(0xFFFF)) | (r[:, _CDP:] << 16)


def _sc_gather_body(table_hbm, idx_hbm, out_hbm, idx_v, *bufs_sems):
    bufs = bufs_sems[:_NBUF]
    sems = bufs_sems[_NBUF:]
    wid = lax.axis_index("s") * 2 + lax.axis_index("c")
    base = wid * _BPW
    row = base // _T
    col = base % _T
    pltpu.sync_copy(idx_hbm.at[row, pl.ds(col, _BPW)], idx_v)
    cps = {}
    for c in range(_NBUF - 1):
        cps[c] = pltpu.async_copy(
            table_hbm.at[idx_v.at[pl.ds(c * _CHUNK, _CHUNK)]],
            bufs[c % _NBUF],
            sems[c % _NBUF],
        )
    for c in range(_NCHUNK):
        cps.pop(c).wait()
        nxt = c + _NBUF - 1
        if nxt < _NCHUNK:
            cps[nxt] = pltpu.async_copy(
                table_hbm.at[idx_v.at[pl.ds(nxt * _CHUNK, _CHUNK)]],
                bufs[nxt % _NBUF],
                sems[nxt % _NBUF],
            )
        pltpu.sync_copy(bufs[c % _NBUF], out_hbm.at[pl.ds(base + c * _CHUNK, _CHUNK)])


_sc_gather = functools.partial(
    pl.kernel,
    mesh=plsc.VectorSubcoreMesh(core_axis_name="c", subcore_axis_name="s"),
    out_type=jax.ShapeDtypeStruct((_N, _CDP), jnp.int32),
    scratch_types=(
        [pltpu.VMEM((_BPW,), jnp.int32)]
        + [pltpu.VMEM((_CHUNK, _CDP), jnp.int32)] * _NBUF
        + [pltpu.SemaphoreType.DMA] * _NBUF
    ),
)(_sc_gather_body)


def _mm_body(x_ref, w_ref, b_ref, o_ref):
    x = x_ref[0]
    xl = lax.bitcast_convert_type(x << 16, jnp.float32)
    xh = lax.bitcast_convert_type(x & jnp.int32(-65536), jnp.float32)
    x2 = jnp.concatenate([xl, xh], axis=1)
    acc = lax.dot_general(
        w_ref[...], x2, (((1,), (1,)), ((), ())), preferred_element_type=jnp.float32
    )
    o_ref[0] = acc + b_ref[...]


def kernel(indices, codebook, W, b):
    packed = pl.pallas_call(
        _pack_body,
        grid=(_K // _PB,),
        in_specs=[pl.BlockSpec((_PB, _CD), lambda i: (i, 0))],
        out_specs=pl.BlockSpec((_PB, _CDP), lambda i: (i, 0)),
        out_shape=jax.ShapeDtypeStruct((_K, _CDP), jnp.int32),
    )(codebook)
    gathered = _sc_gather(packed, indices)
    out = pl.pallas_call(
        _mm_body,
        grid=(_B, _T // _TT),
        in_specs=[
            pl.BlockSpec(
                (1, _TT, _CDP), lambda bb, tt: (bb * (_T // _TT) + tt, 0, 0)
            ),
            pl.BlockSpec((_DIM, _CD), lambda bb, tt: (0, 0)),
            pl.BlockSpec((_DIM, 1), lambda bb, tt: (0, 0)),
        ],
        out_specs=pl.BlockSpec((1, _DIM, _TT), lambda bb, tt: (bb, 0, tt)),
        out_shape=jax.ShapeDtypeStruct((_B, _DIM, _T), jnp.float32),
    )(gathered.reshape(_N // _TT, _TT, _CDP), W, b.reshape(_DIM, 1))
    return out


# speedup vs baseline: 1.0812x; 1.0022x over previous
import functools

import jax
import jax.numpy as jnp
from jax import lax
from jax.experimental import pallas as pl
from jax.experimental.pallas import tpu as pltpu
from jax.experimental.pallas import tpu_sc as plsc

_B, _T, _K, _CD, _DIM = 8, 4096, 8192, 256, 1024
_CDP = _CD // 2
_N = _B * _T
_NW = 32
_BPW = _N // _NW
_CHUNK = 128
_NCHUNK = _BPW // _CHUNK
_NBUF = 4
_PB = 4096
_TT = 4096


def _pack_body(cb_ref, o_ref):
    u = lax.bitcast_convert_type(cb_ref[...], jnp.int32)
    r = (u + jnp.int32(32767) + ((u >> 16) & jnp.int32(1))) >> 16
    o_ref[...] = (r[:, :_CDP] & jnp.int32(0xFFFF)) | (r[:, _CDP:] << 16)


def _sc_gather_body(table_hbm, idx_hbm, out_hbm, idx_v, *bufs_sems):
    bufs = bufs_sems[:_NBUF]
    sems = bufs_sems[_NBUF:]
    wid = lax.axis_index("s") * 2 + lax.axis_index("c")
    base = wid * _BPW
    row = base // _T
    col = base % _T
    pltpu.sync_copy(idx_hbm.at[row, pl.ds(col, _BPW)], idx_v)
    cps = {}
    for c in range(_NBUF - 1):
        cps[c] = pltpu.async_copy(
            table_hbm.at[idx_v.at[pl.ds(c * _CHUNK, _CHUNK)]],
            bufs[c % _NBUF],
            sems[c % _NBUF],
        )
    for c in range(_NCHUNK):
        cps.pop(c).wait()
        nxt = c + _NBUF - 1
        if nxt < _NCHUNK:
            cps[nxt] = pltpu.async_copy(
                table_hbm.at[idx_v.at[pl.ds(nxt * _CHUNK, _CHUNK)]],
                bufs[nxt % _NBUF],
                sems[nxt % _NBUF],
            )
        pltpu.sync_copy(bufs[c % _NBUF], out_hbm.at[pl.ds(base + c * _CHUNK, _CHUNK)])


_sc_gather = functools.partial(
    pl.kernel,
    mesh=plsc.VectorSubcoreMesh(core_axis_name="c", subcore_axis_name="s"),
    out_type=jax.ShapeDtypeStruct((_N, _CDP), jnp.int32),
    scratch_types=(
        [pltpu.VMEM((_BPW,), jnp.int32)]
        + [pltpu.VMEM((_CHUNK, _CDP), jnp.int32)] * _NBUF
        + [pltpu.SemaphoreType.DMA] * _NBUF
    ),
)(_sc_gather_body)


def _mm_body(x_ref, w_ref, b_ref, o_ref):
    x = x_ref[0]
    xl = lax.bitcast_convert_type(x << 16, jnp.float32)
    xh = lax.bitcast_convert_type(x & jnp.int32(-65536), jnp.float32)
    x2 = jnp.concatenate([xl, xh], axis=1)
    acc = lax.dot_general(
        w_ref[...], x2, (((1,), (1,)), ((), ())), preferred_element_type=jnp.float32
    )
    o_ref[0] = acc + b_ref[...]


def kernel(indices, codebook, W, b):
    packed = pl.pallas_call(
        _pack_body,
        grid=(_K // _PB,),
        in_specs=[pl.BlockSpec((_PB, _CD), lambda i: (i, 0))],
        out_specs=pl.BlockSpec((_PB, _CDP), lambda i: (i, 0)),
        out_shape=jax.ShapeDtypeStruct((_K, _CDP), jnp.int32),
    )(codebook)
    gathered = _sc_gather(packed, indices)
    out = pl.pallas_call(
        _mm_body,
        grid=(_B, _T // _TT),
        in_specs=[
            pl.BlockSpec(
                (1, _TT, _CDP), lambda bb, tt: (bb * (_T // _TT) + tt, 0, 0)
            ),
            pl.BlockSpec((_DIM, _CD), lambda bb, tt: (0, 0)),
            pl.BlockSpec((_DIM, 1), lambda bb, tt: (0, 0)),
        ],
        out_specs=pl.BlockSpec((1, _DIM, _TT), lambda bb, tt: (bb, 0, tt)),
        out_shape=jax.ShapeDtypeStruct((_B, _DIM, _T), jnp.float32),
    )(gathered.reshape(_N // _TT, _TT, _CDP), W, b.reshape(_DIM, 1))
    return out
